# baseline (speedup 1.0000x reference)
import jax
import jax.numpy as jnp
from jax.experimental import pallas as pl
from jax.experimental.pallas import tpu as pltpu

_ROWS = 16384
_COLS = 2048
_BR = 256


def _tc_body(p_ref, t_ref, m_ref, o_ref, acc_ref):
    i = pl.program_id(0)

    @pl.when(i == 0)
    def _init():
        acc_ref[0] = 0.0
        acc_ref[1] = 0.0

    m = m_ref[...]
    d = p_ref[...] - t_ref[...]
    acc_ref[0] += jnp.sum(jnp.where(m, d * d, 0.0))
    acc_ref[1] += jnp.sum(jnp.where(m, 1.0, 0.0))

    @pl.when(i == pl.num_programs(0) - 1)
    def _fin():
        o_ref[0] = jnp.sqrt(acc_ref[0] / acc_ref[1])


def kernel(y_pred, y_true, mask):
    p = y_pred.reshape(_ROWS, _COLS)
    t = y_true.reshape(_ROWS, _COLS)
    m = mask.reshape(_ROWS, _COLS)
    out = pl.pallas_call(
        _tc_body,
        grid=(_ROWS // _BR,),
        in_specs=[
            pl.BlockSpec((_BR, _COLS), lambda i: (i, 0)),
            pl.BlockSpec((_BR, _COLS), lambda i: (i, 0)),
            pl.BlockSpec((_BR, _COLS), lambda i: (i, 0)),
        ],
        out_specs=pl.BlockSpec(memory_space=pltpu.SMEM),
        out_shape=jax.ShapeDtypeStruct((1,), jnp.float32),
        scratch_shapes=[pltpu.SMEM((2,), jnp.float32)],
        compiler_params=pltpu.CompilerParams(
            dimension_semantics=("arbitrary",),
        ),
    )(p, t, m)
    return out[0]


# speedup vs baseline: 1.5664x; 1.5664x over previous
import jax
import jax.numpy as jnp
from jax.experimental import pallas as pl
from jax.experimental.pallas import tpu as pltpu

_ROWS = 16384
_COLS = 2048
_MCOLS = _COLS // 4
_BR = 512


def _tc_body(p_ref, t_ref, m_ref, o_ref, acc_ref):
    i = pl.program_id(0)

    @pl.when(i == 0)
    def _init():
        acc_ref[0] = 0.0
        acc_ref[1] = 0.0

    m = m_ref[...].astype(jnp.float32)
    d = p_ref[...] - t_ref[...]
    acc_ref[0] += jnp.sum(d * d * m)
    acc_ref[1] += jnp.sum(m)

    @pl.when(i == pl.num_programs(0) - 1)
    def _fin():
        o_ref[0] = jnp.sqrt(acc_ref[0] / acc_ref[1])


def kernel(y_pred, y_true, mask):
    p = y_pred.reshape(_ROWS, _COLS)
    t = y_true.reshape(_ROWS, _COLS)
    m32 = mask.view(jnp.int8).reshape(_ROWS, _COLS)
    out = pl.pallas_call(
        _tc_body,
        grid=(_ROWS // _BR,),
        in_specs=[
            pl.BlockSpec((_BR, _COLS), lambda i: (i, 0)),
            pl.BlockSpec((_BR, _COLS), lambda i: (i, 0)),
            pl.BlockSpec((_BR, _COLS), lambda i: (i, 0)),
        ],
        out_specs=pl.BlockSpec(memory_space=pltpu.SMEM),
        out_shape=jax.ShapeDtypeStruct((1,), jnp.float32),
        scratch_shapes=[pltpu.SMEM((2,), jnp.float32)],
        compiler_params=pltpu.CompilerParams(
            dimension_semantics=("arbitrary",),
        ),
    )(p, t, m32)
    return out[0]
